# baseline (speedup 1.0000x reference)
import functools

import jax
import jax.numpy as jnp
from jax import lax
from jax.experimental import pallas as pl
from jax.experimental.pallas import tpu as pltpu
from jax.experimental.pallas import tpu_sc as plsc

F32 = jnp.float32
I32 = jnp.int32

AN = 10000
AE = 160000
BNR = 2500
BN = 2560
BE = 40000
BATCH = 16
AEP = 163840
BEP = 40960
NC, NS, L = 2, 16, 16


def _chunk_plan(rows, step=128):
    plan = []
    off = 0
    while off < rows:
        n = min(step, rows - off)
        plan.append((off, n))
        off += n
    return plan


def _make_d_kernel(nv, chw):
    mesh = plsc.VectorSubcoreMesh(core_axis_name="c", subcore_axis_name="s")

    @functools.partial(
        pl.kernel, mesh=mesh,
        out_type=jax.ShapeDtypeStruct((32, chw, 128), F32),
        scratch_types=[
            pltpu.VMEM((3 * nv,), F32),
            pltpu.VMEM((chw, 128), I32),
            pltpu.VMEM((chw, 128), I32),
            pltpu.VMEM((chw, 128), F32),
        ],
        compiler_params=pltpu.CompilerParams(needs_layout_passes=False),
    )
    def dkern(x_hbm, src_hbm, dst_hbm, dout_hbm, xv, sv, dv, ov):
        c = lax.axis_index("c")
        s = lax.axis_index("s")
        w = s * NC + c
        pltpu.sync_copy(x_hbm, xv)
        pltpu.sync_copy(src_hbm.at[w], sv)
        pltpu.sync_copy(dst_hbm.at[w], dv)
        nvm1 = jnp.full((16,), nv - 1, I32)

        def step(t, carry):
            k = t // 8
            j = t % 8
            si = sv[k, pl.ds(j * 16, 16)] * 3
            di = jnp.minimum(dv[k, pl.ds(j * 16, 16)], nvm1) * 3
            e0 = (plsc.load_gather(xv, [si])
                  - plsc.load_gather(xv, [di]) + 1e-8)
            e1 = (plsc.load_gather(xv, [si + 1])
                  - plsc.load_gather(xv, [di + 1]) + 1e-8)
            e2 = (plsc.load_gather(xv, [si + 2])
                  - plsc.load_gather(xv, [di + 2]) + 1e-8)
            s2 = e0 * e0 + e1 * e1 + e2 * e2
            i = plsc.bitcast(s2, I32)
            y = plsc.bitcast(0x5F3759DF - (i >> 1), F32)
            y = y * (1.5 - 0.5 * s2 * y * y)
            y = y * (1.5 - 0.5 * s2 * y * y)
            y = y * (1.5 - 0.5 * s2 * y * y)
            ov[k, pl.ds(j * 16, 16)] = s2 * y
            return carry

        lax.fori_loop(0, chw * 8, step, 0)
        pltpu.sync_copy(ov, dout_hbm.at[w])

    return dkern


def _make_mp_kernel(nv, ch):
    zpt = -(-(nv + 1) // (16 * 8)) * 8
    srows = 16 * zpt
    rpt = (nv // (16 * 8)) * 8
    tail = nv - 16 * rpt
    mesh = plsc.VectorSubcoreMesh(core_axis_name="c", subcore_axis_name="s")

    @functools.partial(
        pl.kernel, mesh=mesh,
        out_type=jax.ShapeDtypeStruct((2 * nv, 128), F32),
        scratch_types=[
            pltpu.VMEM_SHARED((srows, 128), F32),
            pltpu.VMEM((ch, 128), I32),
            pltpu.VMEM((1, 128), I32),
            pltpu.VMEM((1, 128), I32),
            pltpu.VMEM((1, 128), F32),
            pltpu.VMEM((1, 128), F32),
            pltpu.VMEM((128, 128), F32),
            pltpu.VMEM((128, 128), F32),
            pltpu.VMEM((128,), F32),
            pltpu.SemaphoreType.DMA,
            pltpu.SemaphoreType.DMA,
            pltpu.SemaphoreType.DMA,
            pltpu.SemaphoreType.DMA,
            pltpu.SemaphoreType.DMA,
            pltpu.SemaphoreType.DMA,
        ],
        compiler_params=pltpu.CompilerParams(needs_layout_passes=False),
    )
    def mpkern(hw_hbm, pks_hbm, pkd_hbm, dst_hbm, w1d_hbm, agg_hbm,
               aggsh, dvz, ibs0, ibs1, ibd0, ibd1, rows0, rows1, wv,
               gsem0, gsem1, ssem0, ssem1, isem0, isem1):
        c = lax.axis_index("c")
        s = lax.axis_index("s")
        pltpu.async_copy(pks_hbm.at[c, s, 0], ibs0, isem0)
        pltpu.async_copy(pkd_hbm.at[s, 0], ibd0, isem0)
        pltpu.async_copy(pks_hbm.at[c, s, 1], ibs1, isem1)
        pltpu.async_copy(pkd_hbm.at[s, 1], ibd1, isem1)
        pltpu.sync_copy(w1d_hbm.at[c], wv)
        pltpu.sync_copy(dst_hbm.at[s], dvz)

        off = c * nv

        pltpu.make_async_copy(pks_hbm.at[c, s, 0], ibs0, isem0).wait()
        pltpu.make_async_copy(pkd_hbm.at[s, 0], ibd0, isem0).wait()
        pltpu.async_copy(hw_hbm.at[ibs0.at[0]], rows0, gsem0)

        z16 = jnp.zeros((16,), F32)

        def zr(i, carry):
            for j in range(8):
                rows1[i, pl.ds(j * 16, 16)] = z16
            return carry

        lax.fori_loop(0, 128, zr, 0)

        zbase = s * zpt
        for (o, n) in _chunk_plan(zpt):
            pltpu.sync_copy(rows1.at[pl.ds(0, n)], aggsh.at[pl.ds(zbase + o, n)])
        plsc.subcore_barrier()

        wr0 = tuple(wv[pl.ds(j * 16, 16)] for j in range(8))
        bufs = ((rows0, ibs0, ibd0, gsem0, ssem0, isem0),
                (rows1, ibs1, ibd1, gsem1, ssem1, isem1))

        def part(rows, ibd, wr, g0, g1):
            def group(g, wr):
                d16 = ibd[0, pl.ds(g * 16, 16)]
                for t in range(16):
                    r = g * 16 + t
                    dsc = d16[t]
                    for j in range(8):
                        sl = pl.ds(j * 16, 16)
                        rows[r, sl] = jnp.maximum(rows[r, sl] + dsc * wr[j], 0.0)
                return wr

            return lax.fori_loop(g0, g1, group, wr)

        def pairstep(p, wr):
            k0 = p * 2
            for b in range(2):
                cur, ibsc, ibdc, gsem_c, ssem_c, isem_c = bufs[b]
                oth, ibso, ibdo, gsem_o, ssem_o, isem_o = bufs[1 - b]
                k = k0 + b
                pltpu.make_async_copy(hw_hbm.at[ibsc.at[0]], cur, gsem_c).wait()
                wr = part(cur, ibdc, wr, 0, 2)

                @pl.when(k > 0)
                def _():
                    pltpu.make_async_copy(hw_hbm.at[ibsc.at[0]], oth,
                                          ssem_o).wait()

                @pl.when(k + 1 < ch)
                def _():
                    pltpu.make_async_copy(pks_hbm.at[c, s, k + 1], ibso,
                                          isem_o).wait()
                    pltpu.make_async_copy(pkd_hbm.at[s, k + 1], ibdo,
                                          isem_o).wait()
                    pltpu.async_copy(hw_hbm.at[ibso.at[0]], oth, gsem_o)

                wr = part(cur, ibdc, wr, 2, 8)
                pltpu.async_copy(cur, aggsh.at[dvz.at[k]], ssem_c, add=True)

                @pl.when(k + 2 < ch)
                def _():
                    pltpu.async_copy(pks_hbm.at[c, s, k + 2], ibsc, isem_c)
                    pltpu.async_copy(pkd_hbm.at[s, k + 2], ibdc, isem_c)
            return wr

        lax.fori_loop(0, ch // 2, pairstep, wr0)
        pltpu.make_async_copy(hw_hbm.at[ibs1.at[0]], rows1, ssem1).wait()
        plsc.subcore_barrier()

        cbase = s * rpt
        for (o, n) in _chunk_plan(rpt):
            pltpu.sync_copy(aggsh.at[pl.ds(cbase + o, n)],
                            agg_hbm.at[pl.ds(off + cbase + o, n)])
        if tail:
            @pl.when(s == 0)
            def _():
                pltpu.sync_copy(aggsh.at[pl.ds(16 * rpt, tail)],
                                agg_hbm.at[pl.ds(off + 16 * rpt, tail)])

    return mpkern


def _dot(a, b):
    return jnp.dot(a, b, preferred_element_type=F32)


def _embed_body(z_ref, tab_ref, w1_ref, b1_ref, ebb_ref, h0_ref, hw_ref):
    z = z_ref[0, 0, :]
    br = z.shape[0]
    oh = (z[:, None] == lax.broadcasted_iota(I32, (br, 128), 1)).astype(F32)
    h0 = _dot(oh, tab_ref[...])
    h0_ref[...] = h0
    c0 = _dot(ebb_ref[0:1, :16], w1_ref[256:272, :]) + b1_ref[0:1, :]
    hw_ref[0] = _dot(h0, w1_ref[:256, :]) + c0


def _embed_call(Z3, tab, W1p, b1p, ebbp):
    rb, br = Z3.shape[0], Z3.shape[2]
    return pl.pallas_call(
        _embed_body,
        grid=(rb, 2),
        in_specs=[
            pl.BlockSpec((1, 1, br), lambda i, c: (i, 0, 0)),
            pl.BlockSpec((128, 256), lambda i, c: (0, 0)),
            pl.BlockSpec((280, 128), lambda i, c: (0, c)),
            pl.BlockSpec((8, 128), lambda i, c: (0, c)),
            pl.BlockSpec((8, 128), lambda i, c: (0, 0)),
        ],
        out_specs=[
            pl.BlockSpec((br, 256), lambda i, c: (i, 0)),
            pl.BlockSpec((1, br, 128), lambda i, c: (c, i, 0)),
        ],
        out_shape=[
            jax.ShapeDtypeStruct((rb * br, 256), F32),
            jax.ShapeDtypeStruct((2, rb * br, 128), F32),
        ],
    )(Z3, tab, W1p, b1p, ebbp)


def _update_body(h_ref, agg_ref, w2_ref, wn_ref, bn_ref, ebn_ref,
                 hout_ref, hwout_ref):
    aw = _dot(agg_ref[0], w2_ref[:128, :]) + _dot(agg_ref[1], w2_ref[128:, :])
    u = h_ref[...] + aw
    mu = jnp.mean(u, axis=1, keepdims=True)
    dv = u - mu
    var = jnp.mean(dv * dv, axis=1, keepdims=True)
    hn = dv * lax.rsqrt(var + 1e-5)
    hout_ref[...] = hn
    c0 = _dot(ebn_ref[0:1, :16], wn_ref[256:272, :]) + bn_ref[0:1, :]
    hwout_ref[0] = _dot(hn, wn_ref[:256, :]) + c0


def _update_call(h, agg2, W2, Wnp, bnp, ebnp, br):
    nvr = h.shape[0]
    rb = nvr // br
    return pl.pallas_call(
        _update_body,
        grid=(rb, 2),
        in_specs=[
            pl.BlockSpec((br, 256), lambda i, c: (i, 0)),
            pl.BlockSpec((2, br, 128), lambda i, c: (0, i, 0)),
            pl.BlockSpec((256, 256), lambda i, c: (0, 0)),
            pl.BlockSpec((280, 128), lambda i, c: (0, c)),
            pl.BlockSpec((8, 128), lambda i, c: (0, c)),
            pl.BlockSpec((8, 128), lambda i, c: (0, 0)),
        ],
        out_specs=[
            pl.BlockSpec((br, 256), lambda i, c: (i, 0)),
            pl.BlockSpec((1, br, 128), lambda i, c: (c, i, 0)),
        ],
        out_shape=[
            jax.ShapeDtypeStruct((nvr, 256), F32),
            jax.ShapeDtypeStruct((2, nvr, 128), F32),
        ],
    )(h, agg2, W2, Wnp, bnp, ebnp)


def _poolsum_body(h_ref, x_ref, bid_ref, bhs_ref, aux_ref):
    nk = pl.program_id(1)
    bb = pl.program_id(0)
    bid = bid_ref[0, 0, :]
    nr = bid.shape[0]

    @pl.when(nk == 0)
    def _():
        bhs_ref[...] = jnp.zeros_like(bhs_ref)
        aux_ref[...] = jnp.zeros_like(aux_ref)

    lo = bid[0]
    hi = bid[nr - 1]

    @pl.when(jnp.logical_and(hi >= bb * 512, lo < (bb + 1) * 512))
    def _():
        rel = bid - bb * 512
        M = (rel[:, None] == lax.broadcasted_iota(I32, (nr, 512), 1)).astype(F32)
        x4 = x_ref[...] + (lax.broadcasted_iota(I32, (nr, 8), 1) == 3).astype(F32)
        bhs_ref[...] += lax.dot_general(M, h_ref[...], (((0,), (0,)), ((), ())),
                                        preferred_element_type=F32)
        aux_ref[...] += lax.dot_general(M, x4, (((0,), (0,)), ((), ())),
                                        preferred_element_type=F32)


def _poolsum_call(h3, X8, bid3):
    nkc, nr = bid3.shape[0], bid3.shape[2]
    return pl.pallas_call(
        _poolsum_body,
        grid=(BN // 512, nkc),
        in_specs=[
            pl.BlockSpec((nr, 256), lambda bb, nk: (nk, 0)),
            pl.BlockSpec((nr, 8), lambda bb, nk: (nk, 0)),
            pl.BlockSpec((1, 1, nr), lambda bb, nk: (nk, 0, 0)),
        ],
        out_specs=[
            pl.BlockSpec((512, 256), lambda bb, nk: (bb, 0)),
            pl.BlockSpec((512, 8), lambda bb, nk: (bb, 0)),
        ],
        out_shape=[
            jax.ShapeDtypeStruct((BN, 256), F32),
            jax.ShapeDtypeStruct((BN, 8), F32),
        ],
    )(h3, X8, bid3)


def _poolfin_body(bhs_ref, aux_ref, wp_ref, wn_ref, bn_ref, ebn_ref,
                  bh_ref, bx_ref, bhw_ref):
    cnt = jnp.maximum(aux_ref[:, 3:4], 1.0)
    bh = _dot(bhs_ref[...] / cnt, wp_ref[...])
    bh_ref[...] = bh
    bx_ref[...] = aux_ref[...] / cnt
    c0 = _dot(ebn_ref[0:1, :16], wn_ref[256:272, :]) + bn_ref[0:1, :]
    bhw_ref[0] = _dot(bh, wn_ref[:256, :]) + c0


def _poolfin_call(bhs, aux, W_pool, Wtp, btp, ebtp):
    return pl.pallas_call(
        _poolfin_body,
        grid=(1, 2),
        in_specs=[
            pl.BlockSpec((BN, 256), lambda i, c: (0, 0)),
            pl.BlockSpec((BN, 8), lambda i, c: (0, 0)),
            pl.BlockSpec((256, 256), lambda i, c: (0, 0)),
            pl.BlockSpec((280, 128), lambda i, c: (0, c)),
            pl.BlockSpec((8, 128), lambda i, c: (0, c)),
            pl.BlockSpec((8, 128), lambda i, c: (0, 0)),
        ],
        out_specs=[
            pl.BlockSpec((BN, 256), lambda i, c: (0, 0)),
            pl.BlockSpec((BN, 8), lambda i, c: (0, 0)),
            pl.BlockSpec((1, BN, 128), lambda i, c: (c, 0, 0)),
        ],
        out_shape=[
            jax.ShapeDtypeStruct((BN, 256), F32),
            jax.ShapeDtypeStruct((BN, 8), F32),
            jax.ShapeDtypeStruct((2, BN, 128), F32),
        ],
    )(bhs, aux, W_pool, Wtp, btp, ebtp)


def _final_body(bh_ref, bid_ref, h3_ref, wn_ref, tgt_ref, out_ref):
    bid = bid_ref[0, 0, :]
    Mg = (bid[:, None] == lax.broadcasted_iota(I32, (BN, BATCH), 1)).astype(F32)
    gsum = lax.dot_general(Mg, bh_ref[...], (((0,), (0,)), ((), ())),
                           preferred_element_type=F32)
    gcnt = jnp.maximum(jnp.sum(Mg, axis=0)[:, None], 1.0)
    gr = gsum / gcnt
    l2 = jnp.sum(gr * gr) / (BATCH * 256)
    pred = _dot(h3_ref[...], wn_ref[...])
    dfe = pred - tgt_ref[...]
    al = jnp.sum(dfe * dfe) / (AN * 3)
    out_ref[...] = jnp.full((8, 128), al + 0.01 * l2, F32)


def _final_call(bh_top, batch3, h3, wn8, tgt8):
    return pl.pallas_call(
        _final_body,
        grid=(1,),
        in_specs=[
            pl.BlockSpec((BN, 256), lambda i: (0, 0)),
            pl.BlockSpec((1, 1, BN), lambda i: (0, 0, 0)),
            pl.BlockSpec((AN, 256), lambda i: (0, 0)),
            pl.BlockSpec((256, 8), lambda i: (0, 0)),
            pl.BlockSpec((AN, 8), lambda i: (0, 0)),
        ],
        out_specs=pl.BlockSpec((8, 128), lambda i: (0, 0)),
        out_shape=jax.ShapeDtypeStruct((8, 128), F32),
    )(bh_top, batch3, h3, wn8, tgt8)


_d_bottom = _make_d_kernel(AN, AEP // 32 // 128)
_d_top = _make_d_kernel(BN, BEP // 32 // 128)
_mp_bottom = _make_mp_kernel(AN, AEP // 16 // 128)
_mp_top = _make_mp_kernel(BN, BEP // 16 // 128)


def _pack_idx(src, d, nv):
    srcs = src.reshape(16, -1, 1, 128)
    pks = jnp.stack([srcs, srcs + nv])
    pkd = d.reshape(16, -1, 1, 128)
    return pks, pkd


def kernel(Z, X, atom_noise_target, block_id, batch_id, edge_index,
           block_edge_index, atom_embed, edge_embed_bottom, edge_embed_top,
           W1, b1, W2, W_pool, Wt1, bt1, Wt2, W_noise):
    src = jnp.concatenate([edge_index[0], jnp.zeros((AEP - AE,), I32)])
    dst = jnp.concatenate([edge_index[1], jnp.full((AEP - AE,), AN, I32)])
    bsrc = jnp.concatenate([block_edge_index[0], jnp.zeros((BEP - BE,), I32)])
    bdst = jnp.concatenate([block_edge_index[1], jnp.full((BEP - BE,), BN, I32)])
    dst3 = dst.reshape(16, -1, 128)
    bdst3 = bdst.reshape(16, -1, 128)
    src32 = src.reshape(32, -1, 128)
    dst32 = dst.reshape(32, -1, 128)
    bsrc32 = bsrc.reshape(32, -1, 128)
    bdst32 = bdst.reshape(32, -1, 128)
    Z3 = Z.reshape(5, 1, 2000)
    tab = jnp.pad(atom_embed, ((0, 28), (0, 0)))
    W1p = jnp.pad(W1, ((0, 7), (0, 0)))
    Wtp = jnp.pad(Wt1, ((0, 7), (0, 0)))
    b1p = jnp.pad(b1[None, :], ((0, 7), (0, 0)))
    btp = jnp.pad(bt1[None, :], ((0, 7), (0, 0)))
    ebbp = jnp.pad(edge_embed_bottom, ((0, 4), (0, 112)))
    ebtp = jnp.pad(edge_embed_top, ((0, 4), (0, 112)))
    w1d2 = W1[272].reshape(2, 128)
    wt1d2 = Wt1[272].reshape(2, 128)
    X8 = jnp.pad(X, ((0, 0), (0, 5)))
    bid3 = block_id.reshape(5, 1, 2000)
    batch3 = jnp.concatenate(
        [batch_id, jnp.full((BN - BNR,), BATCH, I32)]).reshape(1, 1, BN)
    wn8 = jnp.pad(W_noise, ((0, 0), (0, 5)))
    tgt8 = jnp.pad(atom_noise_target, ((0, 0), (0, 5)))

    d = _d_bottom(X.reshape(-1), src32, dst32).reshape(-1)
    pks, pkd = _pack_idx(src, d, AN)
    h, hw = _embed_call(Z3, tab, W1p, b1p, ebbp)
    for _ in range(3):
        agg = _mp_bottom(hw.reshape(2 * AN, 128), pks, pkd, dst3, w1d2)
        h, hw = _update_call(h, agg.reshape(2, AN, 128), W2, W1p, b1p, ebbp,
                             2000)

    bhs, aux = _poolsum_call(h, X8, bid3)
    bh, bx8, bhw = _poolfin_call(bhs, aux, W_pool, Wtp, btp, ebtp)

    bX = bx8[:, :3]
    bd = _d_top(bX.reshape(-1), bsrc32, bdst32).reshape(-1)
    bpks, bpkd = _pack_idx(bsrc, bd, BN)
    for _ in range(3):
        bagg = _mp_top(bhw.reshape(2 * BN, 128), bpks, bpkd, bdst3, wt1d2)
        bh, bhw = _update_call(bh, bagg.reshape(2, BN, 128), Wt2, Wtp, btp,
                               ebtp, BN)

    out = _final_call(bh, batch3, h, wn8, tgt8)
    return out[0, 0]


# speedup vs baseline: 1.0679x; 1.0679x over previous
import functools

import jax
import jax.numpy as jnp
from jax import lax
from jax.experimental import pallas as pl
from jax.experimental.pallas import tpu as pltpu
from jax.experimental.pallas import tpu_sc as plsc

F32 = jnp.float32
I32 = jnp.int32

AN = 10000
AE = 160000
BNR = 2500
BN = 2560
BE = 40000
BATCH = 16
AEP = 163840
BEP = 40960
NC, NS, L = 2, 16, 16


def _chunk_plan(rows, step=128):
    plan = []
    off = 0
    while off < rows:
        n = min(step, rows - off)
        plan.append((off, n))
        off += n
    return plan


def _make_d_kernel(nv, chw):
    mesh = plsc.VectorSubcoreMesh(core_axis_name="c", subcore_axis_name="s")

    @functools.partial(
        pl.kernel, mesh=mesh,
        out_type=jax.ShapeDtypeStruct((32, chw, 128), F32),
        scratch_types=[
            pltpu.VMEM((3 * nv,), F32),
            pltpu.VMEM((chw, 128), I32),
            pltpu.VMEM((chw, 128), I32),
            pltpu.VMEM((chw, 128), F32),
        ],
        compiler_params=pltpu.CompilerParams(needs_layout_passes=False),
    )
    def dkern(x_hbm, src_hbm, dst_hbm, dout_hbm, xv, sv, dv, ov):
        c = lax.axis_index("c")
        s = lax.axis_index("s")
        w = s * NC + c
        pltpu.sync_copy(x_hbm, xv)
        pltpu.sync_copy(src_hbm.at[w], sv)
        pltpu.sync_copy(dst_hbm.at[w], dv)
        nvm1 = jnp.full((16,), nv - 1, I32)

        def step(t, carry):
            k = t // 8
            j = t % 8
            si = sv[k, pl.ds(j * 16, 16)] * 3
            di = jnp.minimum(dv[k, pl.ds(j * 16, 16)], nvm1) * 3
            e0 = (plsc.load_gather(xv, [si])
                  - plsc.load_gather(xv, [di]) + 1e-8)
            e1 = (plsc.load_gather(xv, [si + 1])
                  - plsc.load_gather(xv, [di + 1]) + 1e-8)
            e2 = (plsc.load_gather(xv, [si + 2])
                  - plsc.load_gather(xv, [di + 2]) + 1e-8)
            s2 = e0 * e0 + e1 * e1 + e2 * e2
            i = plsc.bitcast(s2, I32)
            y = plsc.bitcast(0x5F3759DF - (i >> 1), F32)
            y = y * (1.5 - 0.5 * s2 * y * y)
            y = y * (1.5 - 0.5 * s2 * y * y)
            y = y * (1.5 - 0.5 * s2 * y * y)
            ov[k, pl.ds(j * 16, 16)] = s2 * y
            return carry

        lax.fori_loop(0, chw * 8, step, 0)
        pltpu.sync_copy(ov, dout_hbm.at[w])

    return dkern


def _make_mp_kernel(nv, ch):
    zpt = -(-(nv + 1) // (16 * 8)) * 8
    srows = 16 * zpt
    rpt = (nv // (16 * 8)) * 8
    tail = nv - 16 * rpt
    mesh = plsc.VectorSubcoreMesh(core_axis_name="c", subcore_axis_name="s")

    @functools.partial(
        pl.kernel, mesh=mesh,
        out_type=jax.ShapeDtypeStruct((2 * nv, 128), F32),
        scratch_types=[
            pltpu.VMEM_SHARED((srows, 128), F32),
            pltpu.VMEM((ch, 128), I32),
            pltpu.VMEM((2, 128), I32),
            pltpu.VMEM((2, 128), I32),
            pltpu.VMEM((128, 128), F32),
            pltpu.VMEM((128, 128), F32),
            pltpu.VMEM((128,), F32),
            pltpu.SemaphoreType.DMA,
            pltpu.SemaphoreType.DMA,
            pltpu.SemaphoreType.DMA,
            pltpu.SemaphoreType.DMA,
            pltpu.SemaphoreType.DMA,
            pltpu.SemaphoreType.DMA,
        ],
        compiler_params=pltpu.CompilerParams(needs_layout_passes=False),
    )
    def mpkern(hw_hbm, pk_hbm, dst_hbm, w1d_hbm, agg_hbm,
               aggsh, dvz, ib0, ib1, rows0, rows1, wv,
               gsem0, gsem1, ssem0, ssem1, isem0, isem1):
        c = lax.axis_index("c")
        s = lax.axis_index("s")
        pltpu.async_copy(pk_hbm.at[c, s, 0], ib0, isem0)
        pltpu.async_copy(pk_hbm.at[c, s, 1], ib1, isem1)
        pltpu.sync_copy(w1d_hbm.at[c], wv)
        pltpu.sync_copy(dst_hbm.at[s], dvz)

        off = c * nv

        pltpu.make_async_copy(pk_hbm.at[c, s, 0], ib0, isem0).wait()
        pltpu.async_copy(hw_hbm.at[ib0.at[0]], rows0, gsem0)

        z16 = jnp.zeros((16,), F32)

        def zr(i, carry):
            for j in range(8):
                rows1[i, pl.ds(j * 16, 16)] = z16
            return carry

        lax.fori_loop(0, 128, zr, 0)

        zbase = s * zpt
        for (o, n) in _chunk_plan(zpt):
            pltpu.sync_copy(rows1.at[pl.ds(0, n)], aggsh.at[pl.ds(zbase + o, n)])
        plsc.subcore_barrier()

        wr0 = tuple(wv[pl.ds(j * 16, 16)] for j in range(8))
        bufs = ((rows0, ib0, gsem0, ssem0, isem0),
                (rows1, ib1, gsem1, ssem1, isem1))

        def part(rows, ib, wr, g0, g1):
            def group(g, wr):
                d16 = plsc.bitcast(ib[1, pl.ds(g * 16, 16)], F32)
                for t in range(16):
                    dsc = d16[t]
                    r = g * 16 + t
                    for j in range(8):
                        sl = pl.ds(j * 16, 16)
                        rows[r, sl] = jnp.maximum(rows[r, sl] + dsc * wr[j], 0.0)
                return wr

            return lax.fori_loop(g0, g1, group, wr)

        def pairstep(p, wr):
            k0 = p * 2
            for b in range(2):
                cur, ibc, gsem_c, ssem_c, isem_c = bufs[b]
                oth, ibo, gsem_o, ssem_o, isem_o = bufs[1 - b]
                k = k0 + b
                pltpu.make_async_copy(hw_hbm.at[ibc.at[0]], cur, gsem_c).wait()
                wr = part(cur, ibc, wr, 0, 2)

                @pl.when(k > 0)
                def _():
                    pltpu.make_async_copy(hw_hbm.at[ibc.at[0]], oth,
                                          ssem_o).wait()

                @pl.when(k + 1 < ch)
                def _():
                    pltpu.make_async_copy(pk_hbm.at[c, s, k + 1], ibo,
                                          isem_o).wait()
                    pltpu.async_copy(hw_hbm.at[ibo.at[0]], oth, gsem_o)

                wr = part(cur, ibc, wr, 2, 8)
                pltpu.async_copy(cur, aggsh.at[dvz.at[k]], ssem_c, add=True)

                @pl.when(k + 2 < ch)
                def _():
                    pltpu.async_copy(pk_hbm.at[c, s, k + 2], ibc, isem_c)
            return wr

        lax.fori_loop(0, ch // 2, pairstep, wr0)
        pltpu.make_async_copy(hw_hbm.at[ib1.at[0]], rows1, ssem1).wait()
        plsc.subcore_barrier()

        cbase = s * rpt
        for (o, n) in _chunk_plan(rpt):
            pltpu.sync_copy(aggsh.at[pl.ds(cbase + o, n)],
                            agg_hbm.at[pl.ds(off + cbase + o, n)])
        if tail:
            @pl.when(s == 0)
            def _():
                pltpu.sync_copy(aggsh.at[pl.ds(16 * rpt, tail)],
                                agg_hbm.at[pl.ds(off + 16 * rpt, tail)])

    return mpkern


def _dot(a, b):
    return jnp.dot(a, b, preferred_element_type=F32)


def _embed_body(z_ref, tab_ref, w1_ref, b1_ref, ebb_ref, h0_ref, hw_ref):
    z = z_ref[0, 0, :]
    br = z.shape[0]
    oh = (z[:, None] == lax.broadcasted_iota(I32, (br, 128), 1)).astype(F32)
    h0 = _dot(oh, tab_ref[...])
    h0_ref[...] = h0
    c0 = _dot(ebb_ref[0:1, :16], w1_ref[256:272, :]) + b1_ref[0:1, :]
    hw_ref[0] = _dot(h0, w1_ref[:256, :]) + c0


def _embed_call(Z3, tab, W1p, b1p, ebbp):
    rb, br = Z3.shape[0], Z3.shape[2]
    return pl.pallas_call(
        _embed_body,
        grid=(rb, 2),
        in_specs=[
            pl.BlockSpec((1, 1, br), lambda i, c: (i, 0, 0)),
            pl.BlockSpec((128, 256), lambda i, c: (0, 0)),
            pl.BlockSpec((280, 128), lambda i, c: (0, c)),
            pl.BlockSpec((8, 128), lambda i, c: (0, c)),
            pl.BlockSpec((8, 128), lambda i, c: (0, 0)),
        ],
        out_specs=[
            pl.BlockSpec((br, 256), lambda i, c: (i, 0)),
            pl.BlockSpec((1, br, 128), lambda i, c: (c, i, 0)),
        ],
        out_shape=[
            jax.ShapeDtypeStruct((rb * br, 256), F32),
            jax.ShapeDtypeStruct((2, rb * br, 128), F32),
        ],
    )(Z3, tab, W1p, b1p, ebbp)


def _update_body(h_ref, agg_ref, w2_ref, wn_ref, bn_ref, ebn_ref,
                 hout_ref, hwout_ref):
    aw = _dot(agg_ref[0], w2_ref[:128, :]) + _dot(agg_ref[1], w2_ref[128:, :])
    u = h_ref[...] + aw
    mu = jnp.mean(u, axis=1, keepdims=True)
    dv = u - mu
    var = jnp.mean(dv * dv, axis=1, keepdims=True)
    hn = dv * lax.rsqrt(var + 1e-5)
    hout_ref[...] = hn
    c0 = _dot(ebn_ref[0:1, :16], wn_ref[256:272, :]) + bn_ref[0:1, :]
    hwout_ref[0] = _dot(hn, wn_ref[:256, :]) + c0


def _update_call(h, agg2, W2, Wnp, bnp, ebnp, br):
    nvr = h.shape[0]
    rb = nvr // br
    return pl.pallas_call(
        _update_body,
        grid=(rb, 2),
        in_specs=[
            pl.BlockSpec((br, 256), lambda i, c: (i, 0)),
            pl.BlockSpec((2, br, 128), lambda i, c: (0, i, 0)),
            pl.BlockSpec((256, 256), lambda i, c: (0, 0)),
            pl.BlockSpec((280, 128), lambda i, c: (0, c)),
            pl.BlockSpec((8, 128), lambda i, c: (0, c)),
            pl.BlockSpec((8, 128), lambda i, c: (0, 0)),
        ],
        out_specs=[
            pl.BlockSpec((br, 256), lambda i, c: (i, 0)),
            pl.BlockSpec((1, br, 128), lambda i, c: (c, i, 0)),
        ],
        out_shape=[
            jax.ShapeDtypeStruct((nvr, 256), F32),
            jax.ShapeDtypeStruct((2, nvr, 128), F32),
        ],
    )(h, agg2, W2, Wnp, bnp, ebnp)


def _poolsum_body(h_ref, x_ref, bid_ref, bhs_ref, aux_ref):
    nk = pl.program_id(1)
    bb = pl.program_id(0)
    bid = bid_ref[0, 0, :]
    nr = bid.shape[0]

    @pl.when(nk == 0)
    def _():
        bhs_ref[...] = jnp.zeros_like(bhs_ref)
        aux_ref[...] = jnp.zeros_like(aux_ref)

    lo = bid[0]
    hi = bid[nr - 1]

    @pl.when(jnp.logical_and(hi >= bb * 512, lo < (bb + 1) * 512))
    def _():
        rel = bid - bb * 512
        M = (rel[:, None] == lax.broadcasted_iota(I32, (nr, 512), 1)).astype(F32)
        x4 = x_ref[...] + (lax.broadcasted_iota(I32, (nr, 8), 1) == 3).astype(F32)
        bhs_ref[...] += lax.dot_general(M, h_ref[...], (((0,), (0,)), ((), ())),
                                        preferred_element_type=F32)
        aux_ref[...] += lax.dot_general(M, x4, (((0,), (0,)), ((), ())),
                                        preferred_element_type=F32)


def _poolsum_call(h3, X8, bid3):
    nkc, nr = bid3.shape[0], bid3.shape[2]
    return pl.pallas_call(
        _poolsum_body,
        grid=(BN // 512, nkc),
        in_specs=[
            pl.BlockSpec((nr, 256), lambda bb, nk: (nk, 0)),
            pl.BlockSpec((nr, 8), lambda bb, nk: (nk, 0)),
            pl.BlockSpec((1, 1, nr), lambda bb, nk: (nk, 0, 0)),
        ],
        out_specs=[
            pl.BlockSpec((512, 256), lambda bb, nk: (bb, 0)),
            pl.BlockSpec((512, 8), lambda bb, nk: (bb, 0)),
        ],
        out_shape=[
            jax.ShapeDtypeStruct((BN, 256), F32),
            jax.ShapeDtypeStruct((BN, 8), F32),
        ],
    )(h3, X8, bid3)


def _poolfin_body(bhs_ref, aux_ref, wp_ref, wn_ref, bn_ref, ebn_ref,
                  bh_ref, bx_ref, bhw_ref):
    cnt = jnp.maximum(aux_ref[:, 3:4], 1.0)
    bh = _dot(bhs_ref[...] / cnt, wp_ref[...])
    bh_ref[...] = bh
    bx_ref[...] = aux_ref[...] / cnt
    c0 = _dot(ebn_ref[0:1, :16], wn_ref[256:272, :]) + bn_ref[0:1, :]
    bhw_ref[0] = _dot(bh, wn_ref[:256, :]) + c0


def _poolfin_call(bhs, aux, W_pool, Wtp, btp, ebtp):
    return pl.pallas_call(
        _poolfin_body,
        grid=(1, 2),
        in_specs=[
            pl.BlockSpec((BN, 256), lambda i, c: (0, 0)),
            pl.BlockSpec((BN, 8), lambda i, c: (0, 0)),
            pl.BlockSpec((256, 256), lambda i, c: (0, 0)),
            pl.BlockSpec((280, 128), lambda i, c: (0, c)),
            pl.BlockSpec((8, 128), lambda i, c: (0, c)),
            pl.BlockSpec((8, 128), lambda i, c: (0, 0)),
        ],
        out_specs=[
            pl.BlockSpec((BN, 256), lambda i, c: (0, 0)),
            pl.BlockSpec((BN, 8), lambda i, c: (0, 0)),
            pl.BlockSpec((1, BN, 128), lambda i, c: (c, 0, 0)),
        ],
        out_shape=[
            jax.ShapeDtypeStruct((BN, 256), F32),
            jax.ShapeDtypeStruct((BN, 8), F32),
            jax.ShapeDtypeStruct((2, BN, 128), F32),
        ],
    )(bhs, aux, W_pool, Wtp, btp, ebtp)


def _final_body(bh_ref, bid_ref, h3_ref, wn_ref, tgt_ref, out_ref):
    bid = bid_ref[0, 0, :]
    Mg = (bid[:, None] == lax.broadcasted_iota(I32, (BN, BATCH), 1)).astype(F32)
    gsum = lax.dot_general(Mg, bh_ref[...], (((0,), (0,)), ((), ())),
                           preferred_element_type=F32)
    gcnt = jnp.maximum(jnp.sum(Mg, axis=0)[:, None], 1.0)
    gr = gsum / gcnt
    l2 = jnp.sum(gr * gr) / (BATCH * 256)
    pred = _dot(h3_ref[...], wn_ref[...])
    dfe = pred - tgt_ref[...]
    al = jnp.sum(dfe * dfe) / (AN * 3)
    out_ref[...] = jnp.full((8, 128), al + 0.01 * l2, F32)


def _final_call(bh_top, batch3, h3, wn8, tgt8):
    return pl.pallas_call(
        _final_body,
        grid=(1,),
        in_specs=[
            pl.BlockSpec((BN, 256), lambda i: (0, 0)),
            pl.BlockSpec((1, 1, BN), lambda i: (0, 0, 0)),
            pl.BlockSpec((AN, 256), lambda i: (0, 0)),
            pl.BlockSpec((256, 8), lambda i: (0, 0)),
            pl.BlockSpec((AN, 8), lambda i: (0, 0)),
        ],
        out_specs=pl.BlockSpec((8, 128), lambda i: (0, 0)),
        out_shape=jax.ShapeDtypeStruct((8, 128), F32),
    )(bh_top, batch3, h3, wn8, tgt8)


_d_bottom = _make_d_kernel(AN, AEP // 32 // 128)
_d_top = _make_d_kernel(BN, BEP // 32 // 128)
_mp_bottom = _make_mp_kernel(AN, AEP // 16 // 128)
_mp_top = _make_mp_kernel(BN, BEP // 16 // 128)


def _pack_idx(src, d, nv):
    srcs = src.reshape(16, -1, 128)
    di = lax.bitcast_convert_type(d, I32).reshape(16, -1, 128)
    return jnp.stack([jnp.stack([srcs, di], axis=2),
                      jnp.stack([srcs + nv, di], axis=2)])


def kernel(Z, X, atom_noise_target, block_id, batch_id, edge_index,
           block_edge_index, atom_embed, edge_embed_bottom, edge_embed_top,
           W1, b1, W2, W_pool, Wt1, bt1, Wt2, W_noise):
    src = jnp.concatenate([edge_index[0], jnp.zeros((AEP - AE,), I32)])
    dst = jnp.concatenate([edge_index[1], jnp.full((AEP - AE,), AN, I32)])
    bsrc = jnp.concatenate([block_edge_index[0], jnp.zeros((BEP - BE,), I32)])
    bdst = jnp.concatenate([block_edge_index[1], jnp.full((BEP - BE,), BN, I32)])
    dst3 = dst.reshape(16, -1, 128)
    bdst3 = bdst.reshape(16, -1, 128)
    src32 = src.reshape(32, -1, 128)
    dst32 = dst.reshape(32, -1, 128)
    bsrc32 = bsrc.reshape(32, -1, 128)
    bdst32 = bdst.reshape(32, -1, 128)
    Z3 = Z.reshape(5, 1, 2000)
    tab = jnp.pad(atom_embed, ((0, 28), (0, 0)))
    W1p = jnp.pad(W1, ((0, 7), (0, 0)))
    Wtp = jnp.pad(Wt1, ((0, 7), (0, 0)))
    b1p = jnp.pad(b1[None, :], ((0, 7), (0, 0)))
    btp = jnp.pad(bt1[None, :], ((0, 7), (0, 0)))
    ebbp = jnp.pad(edge_embed_bottom, ((0, 4), (0, 112)))
    ebtp = jnp.pad(edge_embed_top, ((0, 4), (0, 112)))
    w1d2 = W1[272].reshape(2, 128)
    wt1d2 = Wt1[272].reshape(2, 128)
    X8 = jnp.pad(X, ((0, 0), (0, 5)))
    bid3 = block_id.reshape(5, 1, 2000)
    batch3 = jnp.concatenate(
        [batch_id, jnp.full((BN - BNR,), BATCH, I32)]).reshape(1, 1, BN)
    wn8 = jnp.pad(W_noise, ((0, 0), (0, 5)))
    tgt8 = jnp.pad(atom_noise_target, ((0, 0), (0, 5)))

    d = _d_bottom(X.reshape(-1), src32, dst32).reshape(-1)
    pk = _pack_idx(src, d, AN)
    h, hw = _embed_call(Z3, tab, W1p, b1p, ebbp)
    for _ in range(3):
        agg = _mp_bottom(hw.reshape(2 * AN, 128), pk, dst3, w1d2)
        h, hw = _update_call(h, agg.reshape(2, AN, 128), W2, W1p, b1p, ebbp,
                             2000)

    bhs, aux = _poolsum_call(h, X8, bid3)
    bh, bx8, bhw = _poolfin_call(bhs, aux, W_pool, Wtp, btp, ebtp)

    bX = bx8[:, :3]
    bd = _d_top(bX.reshape(-1), bsrc32, bdst32).reshape(-1)
    bpk = _pack_idx(bsrc, bd, BN)
    for _ in range(3):
        bagg = _mp_top(bhw.reshape(2 * BN, 128), bpk, bdst3, wt1d2)
        bh, bhw = _update_call(bh, bagg.reshape(2, BN, 128), Wt2, Wtp, btp,
                               ebtp, BN)

    out = _final_call(bh, batch3, h, wn8, tgt8)
    return out[0, 0]
